# baseline (speedup 1.0000x reference)
import functools
import math

import numpy as np
import jax
import jax.numpy as jnp
from jax import lax
from jax.experimental import pallas as pl
from jax.experimental.pallas import tpu as pltpu
from jax.experimental.pallas import tpu_sc as plsc

VOCAB = 100000
DIM = 128
MAX_LEN = 200
BATCH = 1024
SEQ = 200

N = BATCH * SEQ
NW = 32
ROWS_PER_W = N // NW
CHUNK = 128
NCHUNK = ROWS_PER_W // CHUNK


def _pe_table_np():
    position = np.arange(MAX_LEN, dtype=np.float32)[:, None]
    div_term = np.exp(
        np.arange(0, DIM, 2, dtype=np.float32) * -(math.log(10000.0) / DIM)
    )
    pe = np.zeros((MAX_LEN, DIM), dtype=np.float32)
    pe[:, 0::2] = np.sin(position * div_term)
    pe[:, 1::2] = np.cos(position * div_term)
    return pe


_PE_NP = _pe_table_np()


def _prep_body(seg_ref, s_ref, pe_ref, comb_ref, cidx_ref):
    pe = pe_ref[...]
    s = s_ref[...]
    comb_ref[...] = pe[None, :, :] + s[:, None, :]
    pos = lax.broadcasted_iota(jnp.int32, (BATCH, SEQ), 1)
    cidx_ref[...] = seg_ref[...] * SEQ + pos


@jax.jit
def _prep(segment, s_table, pe):
    return pl.pallas_call(
        _prep_body,
        out_shape=(
            jax.ShapeDtypeStruct((3, SEQ, DIM), jnp.float32),
            jax.ShapeDtypeStruct((BATCH, SEQ), jnp.int32),
        ),
    )(segment, s_table, pe)


NSLOT = 5


def _sc_body(t_hbm, comb_hbm, tok_hbm, cidx_hbm, out_hbm,
             comb_sh, idx_t, idx_c, bufs, sems_c, sems_t, sems_w):
    cid = lax.axis_index("c")
    sid = lax.axis_index("s")
    wid = sid * 2 + cid
    obase = wid * ROWS_PER_W

    @pl.when(sid == 0)
    def _():
        pltpu.sync_copy(comb_hbm, comb_sh)

    pltpu.sync_copy(tok_hbm.at[pl.ds(obase, ROWS_PER_W)], idx_t)
    pltpu.sync_copy(cidx_hbm.at[pl.ds(obase, ROWS_PER_W)], idx_c)
    plsc.subcore_barrier()

    def wb_drain(b):
        pltpu.make_async_copy(
            bufs[b], out_hbm.at[pl.ds(obase, CHUNK)], sems_w[b]
        ).wait()

    def group_body(gg, carry):
        g0 = gg * NSLOT

        def islice(ref, b):
            off = pl.multiple_of((g0 + b) * CHUNK, CHUNK)
            return ref.at[pl.ds(off, CHUNK)]

        tds = []
        for b in range(NSLOT):
            @pl.when(gg > 0)
            def _(b=b):
                wb_drain(b)
            tds.append(
                pltpu.async_copy(t_hbm.at[islice(idx_t, b)], bufs[b],
                                 sems_t[b])
            )
        cds = []
        for b in range(NSLOT):
            tds[b].wait()
            cds.append(
                pltpu.async_copy(comb_sh.at[islice(idx_c, b)], bufs[b],
                                 sems_c[b], add=True)
            )
        for b in range(NSLOT):
            cds[b].wait()
            pltpu.async_copy(
                bufs[b],
                out_hbm.at[pl.ds(obase + (g0 + b) * CHUNK, CHUNK)],
                sems_w[b],
            )
        return carry

    lax.fori_loop(0, NCHUNK // NSLOT, group_body, 0)
    for b in range(NSLOT):
        wb_drain(b)


@jax.jit
def _sc_gather(t_table, comb, tok, cidx):
    mesh = plsc.VectorSubcoreMesh(core_axis_name="c", subcore_axis_name="s")
    f = pl.kernel(
        _sc_body,
        out_type=jax.ShapeDtypeStruct((N, DIM), jnp.float32),
        mesh=mesh,
        scratch_types=[
            pltpu.VMEM_SHARED((3 * SEQ, DIM), jnp.float32),
            pltpu.VMEM((ROWS_PER_W,), jnp.int32),
            pltpu.VMEM((ROWS_PER_W,), jnp.int32),
            [pltpu.VMEM((CHUNK, DIM), jnp.float32) for _ in range(NSLOT)],
            [pltpu.SemaphoreType.DMA for _ in range(NSLOT)],
            [pltpu.SemaphoreType.DMA for _ in range(NSLOT)],
            [pltpu.SemaphoreType.DMA for _ in range(NSLOT)],
        ],
    )
    return f(t_table, comb, tok, cidx)


def kernel(input_batch, segment, t_table, s_table):
    comb3, cidx = _prep(segment, s_table, _PE_NP)
    comb = comb3.reshape(3 * SEQ, DIM)
    tok = input_batch.reshape(-1)
    cidx_flat = cidx.reshape(-1)
    out = _sc_gather(t_table, comb, tok, cidx_flat)
    return out.reshape(BATCH, SEQ, DIM)


# speedup vs baseline: 1.0466x; 1.0466x over previous
import functools
import math

import numpy as np
import jax
import jax.numpy as jnp
from jax import lax
from jax.experimental import pallas as pl
from jax.experimental.pallas import tpu as pltpu
from jax.experimental.pallas import tpu_sc as plsc

VOCAB = 100000
DIM = 128
MAX_LEN = 200
BATCH = 1024
SEQ = 200

N = BATCH * SEQ
NW = 32
ROWS_PER_W = N // NW
CHUNK = 128
NCHUNK = ROWS_PER_W // CHUNK


def _pe_table_np():
    position = np.arange(MAX_LEN, dtype=np.float32)[:, None]
    div_term = np.exp(
        np.arange(0, DIM, 2, dtype=np.float32) * -(math.log(10000.0) / DIM)
    )
    pe = np.zeros((MAX_LEN, DIM), dtype=np.float32)
    pe[:, 0::2] = np.sin(position * div_term)
    pe[:, 1::2] = np.cos(position * div_term)
    return pe


_PE_NP = _pe_table_np()


def _prep_body(seg_ref, s_ref, pe_ref, comb_ref, cidx_ref):
    pe = pe_ref[...]
    s = s_ref[...]
    comb_ref[...] = pe[None, :, :] + s[:, None, :]
    pos = lax.broadcasted_iota(jnp.int32, (BATCH, SEQ), 1)
    cidx_ref[...] = seg_ref[...] * SEQ + pos


@jax.jit
def _prep(segment, s_table, pe):
    return pl.pallas_call(
        _prep_body,
        out_shape=(
            jax.ShapeDtypeStruct((3, SEQ, DIM), jnp.float32),
            jax.ShapeDtypeStruct((BATCH, SEQ), jnp.int32),
        ),
    )(segment, s_table, pe)


NSLOT = 5


def _sc_body(t_hbm, comb_hbm, tok_hbm, cidx_hbm, out_hbm,
             comb_sh, idx_t, idx_c, bufs, sems_c, sems_t, sems_w):
    cid = lax.axis_index("c")
    sid = lax.axis_index("s")
    wid = sid * 2 + cid
    obase = wid * ROWS_PER_W

    @pl.when(sid == 0)
    def _():
        pltpu.sync_copy(comb_hbm, comb_sh)

    pltpu.sync_copy(tok_hbm.at[pl.ds(obase, ROWS_PER_W)], idx_t)
    pltpu.sync_copy(cidx_hbm.at[pl.ds(obase, ROWS_PER_W)], idx_c)
    plsc.subcore_barrier()

    def wb_drain(b):
        pltpu.make_async_copy(
            bufs[b], out_hbm.at[pl.ds(obase, CHUNK)], sems_w[b]
        ).wait()

    def group_body(gg, carry):
        g0 = gg * NSLOT

        def islice(ref, b):
            off = pl.multiple_of((g0 + b) * CHUNK, CHUNK)
            return ref.at[pl.ds(off, CHUNK)]

        cds = []
        for b in range(NSLOT):
            @pl.when(gg > 0)
            def _(b=b):
                wb_drain(b)
            cds.append(
                pltpu.async_copy(comb_sh.at[islice(idx_c, b)], bufs[b],
                                 sems_c[b])
            )
        tds = []
        for b in range(NSLOT):
            cds[b].wait()
            tds.append(
                pltpu.async_copy(t_hbm.at[islice(idx_t, b)], bufs[b],
                                 sems_t[b], add=True)
            )
        for b in range(NSLOT):
            tds[b].wait()
            pltpu.async_copy(
                bufs[b],
                out_hbm.at[pl.ds(obase + (g0 + b) * CHUNK, CHUNK)],
                sems_w[b],
            )
        return carry

    lax.fori_loop(0, NCHUNK // NSLOT, group_body, 0)
    for b in range(NSLOT):
        wb_drain(b)


@jax.jit
def _sc_gather(t_table, comb, tok, cidx):
    mesh = plsc.VectorSubcoreMesh(core_axis_name="c", subcore_axis_name="s")
    f = pl.kernel(
        _sc_body,
        out_type=jax.ShapeDtypeStruct((N, DIM), jnp.float32),
        mesh=mesh,
        scratch_types=[
            pltpu.VMEM_SHARED((3 * SEQ, DIM), jnp.float32),
            pltpu.VMEM((ROWS_PER_W,), jnp.int32),
            pltpu.VMEM((ROWS_PER_W,), jnp.int32),
            [pltpu.VMEM((CHUNK, DIM), jnp.float32) for _ in range(NSLOT)],
            [pltpu.SemaphoreType.DMA for _ in range(NSLOT)],
            [pltpu.SemaphoreType.DMA for _ in range(NSLOT)],
            [pltpu.SemaphoreType.DMA for _ in range(NSLOT)],
        ],
    )
    return f(t_table, comb, tok, cidx)


def kernel(input_batch, segment, t_table, s_table):
    comb3, cidx = _prep(segment, s_table, _PE_NP)
    comb = comb3.reshape(3 * SEQ, DIM)
    tok = input_batch.reshape(-1)
    cidx_flat = cidx.reshape(-1)
    out = _sc_gather(t_table, comb, tok, cidx_flat)
    return out.reshape(BATCH, SEQ, DIM)
